# baseline (speedup 1.0000x reference)
import functools

import jax
import jax.numpy as jnp
from jax.experimental import pallas as pl
from jax.experimental.pallas import tpu as pltpu
from jax.experimental.pallas import tpu_sc as plsc

_D = 64
_NB = 16


def _lookup(token_ids, weight):
    batch, hist = token_ids.shape
    mesh = plsc.VectorSubcoreMesh(core_axis_name="core", subcore_axis_name="subcore")

    @functools.partial(
        pl.kernel,
        out_type=jax.ShapeDtypeStruct((batch, hist, _D), weight.dtype),
        mesh=mesh,
        scratch_types=[pltpu.SemaphoreType.DMA],
        compiler_params=pltpu.CompilerParams(use_tc_tiling_on_sc=False),
    )
    def k(w_hbm, i_hbm, o_hbm, sem):
        def body(i_vmem, o_vmem):
            copies = [
                pltpu.async_copy(
                    w_hbm.at[i_vmem.at[j]],
                    o_vmem.at[j],
                    sem,
                )
                for j in range(_NB)
            ]
            for c in copies:
                c.wait()

        pltpu.emit_pipeline(
            body,
            grid=(batch // _NB,),
            in_specs=[pl.BlockSpec((_NB, hist), index_map=lambda i: (i, 0))],
            out_specs=[pl.BlockSpec((_NB, hist, _D), index_map=lambda i: (i, 0, 0))],
            core_axis_name=("core", "subcore"),
            dimension_semantics=(pltpu.PARALLEL,),
        )(i_hbm, o_hbm)

    return k(weight, token_ids)


def kernel(token_ids, weight):
    return _lookup(token_ids, weight)
